# baseline (speedup 1.0000x reference)
import jax
import jax.numpy as jnp
from jax import lax
from jax.experimental import pallas as pl
from jax.experimental.pallas import tpu as pltpu
from jax.experimental.pallas import tpu_sc as plsc

_N = 8192
_F = 4096
_CLS = 100
_PAD = 112
_NC = 2
_NS = 16
_NT = _NC * _NS
_FT = _F // _NT
_CH = 16
_NCHUNK = _N // _CH
_NPAIR = _NCHUNK // 2
_OWN = _N // _NT
_L = 16


def _sc_body(xs_hbm, lab_hbm, sum_out, aux_out,
             acc_sum, acc_aux, xv0, xv1, lv_all, sem0, sem1):
    cid = lax.axis_index("c")
    sid = lax.axis_index("s")
    tile = cid * _NS + sid

    zero16 = jnp.zeros((_L,), jnp.float32)
    one16 = jnp.ones((_L,), jnp.float32)
    iota16 = lax.iota(jnp.int32, _L)

    pltpu.sync_copy(lab_hbm, lv_all)

    def _xsrc(it):
        off = pl.multiple_of(tile * _N * _FT + it * _CH * _FT, _CH * _FT)
        return xs_hbm.at[pl.ds(off, _CH * _FT)]

    pltpu.async_copy(_xsrc(0), xv0, sem0)

    def _zacc(r, carry):
        for g in range(_FT // _L):
            acc_sum[r, pl.ds(g * _L, _L)] = zero16
            acc_aux[r, pl.ds(g * _L, _L)] = zero16
        return carry
    lax.fori_loop(0, _PAD, _zacc, None)

    def _bcast(lvv, r):
        return lax.gather(
            lvv, jnp.full((_L, 1), r, jnp.int32),
            dimension_numbers=lax.GatherDimensionNumbers(
                offset_dims=(), collapsed_slice_dims=(0,),
                start_index_map=(0,)),
            slice_sizes=(1,),
            mode=lax.GatherScatterMode.PROMISE_IN_BOUNDS)

    def _compute(it, xv):
        base = it * _CH
        ng = _FT // _L

        def _stage(row):
            lvv = lv_all[pl.ds(base + (row // _L) * _L, _L)]
            lab16 = _bcast(lvv, row % _L)
            vs = [xv[pl.ds(row * _FT + g * _L, _L)] for g in range(ng)]
            sq0 = (vs[0] * vs[0] + vs[2] * vs[2]
                   + vs[4] * vs[4] + vs[6] * vs[6])
            sq1 = (vs[1] * vs[1] + vs[3] * vs[3]
                   + vs[5] * vs[5] + vs[7] * vs[7])
            return lab16, vs, sq0 + sq1

        cur = _stage(0)
        for row in range(_CH):
            nxt = _stage(row + 1) if row + 1 < _CH else None
            lab16, vs, sq = cur
            for g in range(ng):
                plsc.addupdate_scatter(acc_sum,
                                       [lab16, iota16 + (g * _L)], vs[g])
            plsc.addupdate_scatter(acc_aux, [lab16, iota16], sq)
            cur = nxt

    def _pair(i, carry):
        it0 = 2 * i
        pltpu.make_async_copy(_xsrc(it0), xv0, sem0).wait()
        pltpu.async_copy(_xsrc(it0 + 1), xv1, sem1)
        _compute(it0, xv0)
        pltpu.make_async_copy(_xsrc(it0 + 1), xv1, sem1).wait()

        @pl.when(i < _NPAIR - 1)
        def _prefetch():
            pltpu.async_copy(_xsrc(it0 + 2), xv0, sem0)
        _compute(it0 + 1, xv1)
        return carry
    lax.fori_loop(0, _NPAIR, _pair, None)

    own0 = tile * _OWN
    for s in range(_OWN // _L):
        lvv = lv_all[pl.ds(own0 + s * _L, _L)]
        plsc.addupdate_scatter(acc_aux, [lvv, iota16 + _L], one16)

    pltpu.sync_copy(acc_sum, sum_out.at[tile])
    pltpu.sync_copy(acc_aux, aux_out.at[tile])


def _tc_epilogue(sum_ref, aux_ref, out_ref):
    s = sum_ref[...]
    norm2 = jnp.sum(jnp.sum(s * s, axis=2), axis=0)
    aux = jnp.sum(aux_ref[...], axis=0)
    sumsq = jnp.sum(aux[:, :_L], axis=1)
    n = jnp.sum(aux[:, _L:2 * _L], axis=1)
    sqc = sumsq - norm2 / jnp.maximum(n, 1.0)
    norm = jnp.where((n > 0.0) & (sqc > 0.0),
                     jnp.sqrt(jnp.maximum(sqc, 1e-30)), 0.0)
    out_ref[...] = jnp.reshape(jnp.sum(norm) / _CLS, (1, 1))


def kernel(x, labels):
    xs = jnp.transpose(x.reshape(_N, _NT, _FT), (1, 0, 2)).reshape(-1)
    lab = labels.astype(jnp.int32)

    sc_call = pl.kernel(
        _sc_body,
        out_type=(
            jax.ShapeDtypeStruct((_NT, _PAD, _FT), jnp.float32),
            jax.ShapeDtypeStruct((_NT, _PAD, _FT), jnp.float32),
        ),
        mesh=plsc.VectorSubcoreMesh(core_axis_name="c", subcore_axis_name="s",
                                    num_cores=_NC, num_subcores=_NS),
        compiler_params=pltpu.CompilerParams(needs_layout_passes=False),
        scratch_types=[
            pltpu.VMEM((_PAD, _FT), jnp.float32),
            pltpu.VMEM((_PAD, _FT), jnp.float32),
            pltpu.VMEM((_CH * _FT,), jnp.float32),
            pltpu.VMEM((_CH * _FT,), jnp.float32),
            pltpu.VMEM((_N,), jnp.int32),
            pltpu.SemaphoreType.DMA,
            pltpu.SemaphoreType.DMA,
        ],
    )
    sums, aux = sc_call(xs, lab)

    out = pl.pallas_call(
        _tc_epilogue,
        out_shape=jax.ShapeDtypeStruct((1, 1), jnp.float32),
    )(sums, aux)
    return out[0, 0]


# speedup vs baseline: 1.3217x; 1.3217x over previous
import jax
import jax.numpy as jnp
from jax import lax
from jax.experimental import pallas as pl
from jax.experimental.pallas import tpu as pltpu
from jax.experimental.pallas import tpu_sc as plsc

_N = 8192
_F = 4096
_CLS = 100
_PAD = 112
_NC = 2
_NS = 16
_NT = _NC * _NS
_FT = _F // _NT
_CH = 32
_NCHUNK = _N // _CH
_NPAIR = _NCHUNK // 2
_OWN = _N // _NT
_L = 16


def _sc_body(xs_hbm, lab_hbm, sum_out, aux_out,
             acc_sum, acc_aux, xv0, xv1, lv_all, sem0, sem1):
    cid = lax.axis_index("c")
    sid = lax.axis_index("s")
    tile = cid * _NS + sid

    zero16 = jnp.zeros((_L,), jnp.float32)
    one16 = jnp.ones((_L,), jnp.float32)
    iota16 = lax.iota(jnp.int32, _L)

    pltpu.sync_copy(lab_hbm, lv_all)

    def _xsrc(it):
        off = pl.multiple_of(tile * _N * _FT + it * _CH * _FT, _CH * _FT)
        return xs_hbm.at[pl.ds(off, _CH * _FT)]

    pltpu.async_copy(_xsrc(0), xv0, sem0)

    def _zacc(r, carry):
        for g in range(_FT // _L):
            acc_sum[r, pl.ds(g * _L, _L)] = zero16
            acc_aux[r, pl.ds(g * _L, _L)] = zero16
        return carry
    lax.fori_loop(0, _PAD, _zacc, None)

    def _bcast(lvv, r):
        return lax.gather(
            lvv, jnp.full((_L, 1), r, jnp.int32),
            dimension_numbers=lax.GatherDimensionNumbers(
                offset_dims=(), collapsed_slice_dims=(0,),
                start_index_map=(0,)),
            slice_sizes=(1,),
            mode=lax.GatherScatterMode.PROMISE_IN_BOUNDS)

    def _compute(it, xv):
        base = it * _CH
        ng = _FT // _L

        def _stage(row):
            lvv = lv_all[pl.ds(base + (row // _L) * _L, _L)]
            lab16 = _bcast(lvv, row % _L)
            vs = [xv[pl.ds(row * _FT + g * _L, _L)] for g in range(ng)]
            sq0 = (vs[0] * vs[0] + vs[2] * vs[2]
                   + vs[4] * vs[4] + vs[6] * vs[6])
            sq1 = (vs[1] * vs[1] + vs[3] * vs[3]
                   + vs[5] * vs[5] + vs[7] * vs[7])
            return lab16, vs, sq0 + sq1

        cur = _stage(0)
        for row in range(_CH):
            nxt = _stage(row + 1) if row + 1 < _CH else None
            lab16, vs, sq = cur
            for g in range(ng):
                plsc.addupdate_scatter(acc_sum,
                                       [lab16, iota16 + (g * _L)], vs[g])
            plsc.addupdate_scatter(acc_aux, [lab16, iota16], sq)
            cur = nxt

    def _pair(i, carry):
        it0 = 2 * i
        pltpu.make_async_copy(_xsrc(it0), xv0, sem0).wait()
        pltpu.async_copy(_xsrc(it0 + 1), xv1, sem1)
        _compute(it0, xv0)
        pltpu.make_async_copy(_xsrc(it0 + 1), xv1, sem1).wait()

        @pl.when(i < _NPAIR - 1)
        def _prefetch():
            pltpu.async_copy(_xsrc(it0 + 2), xv0, sem0)
        _compute(it0 + 1, xv1)
        return carry
    lax.fori_loop(0, _NPAIR, _pair, None)

    own0 = tile * _OWN
    for s in range(_OWN // _L):
        lvv = lv_all[pl.ds(own0 + s * _L, _L)]
        plsc.addupdate_scatter(acc_aux, [lvv, iota16 + _L], one16)

    pltpu.sync_copy(acc_sum, sum_out.at[tile])
    pltpu.sync_copy(acc_aux, aux_out.at[tile])


def _tc_epilogue(sum_ref, aux_ref, out_ref):
    s = sum_ref[...]
    norm2 = jnp.sum(jnp.sum(s * s, axis=2), axis=0)
    aux = jnp.sum(aux_ref[...], axis=0)
    sumsq = jnp.sum(aux[:, :_L], axis=1)
    n = jnp.sum(aux[:, _L:2 * _L], axis=1)
    sqc = sumsq - norm2 / jnp.maximum(n, 1.0)
    norm = jnp.where((n > 0.0) & (sqc > 0.0),
                     jnp.sqrt(jnp.maximum(sqc, 1e-30)), 0.0)
    out_ref[...] = jnp.reshape(jnp.sum(norm) / _CLS, (1, 1))


def kernel(x, labels):
    xs = jnp.transpose(x.reshape(_N, _NT, _FT), (1, 0, 2)).reshape(-1)
    lab = labels.astype(jnp.int32)

    sc_call = pl.kernel(
        _sc_body,
        out_type=(
            jax.ShapeDtypeStruct((_NT, _PAD, _FT), jnp.float32),
            jax.ShapeDtypeStruct((_NT, _PAD, _FT), jnp.float32),
        ),
        mesh=plsc.VectorSubcoreMesh(core_axis_name="c", subcore_axis_name="s",
                                    num_cores=_NC, num_subcores=_NS),
        compiler_params=pltpu.CompilerParams(needs_layout_passes=False),
        scratch_types=[
            pltpu.VMEM((_PAD, _FT), jnp.float32),
            pltpu.VMEM((_PAD, _FT), jnp.float32),
            pltpu.VMEM((_CH * _FT,), jnp.float32),
            pltpu.VMEM((_CH * _FT,), jnp.float32),
            pltpu.VMEM((_N,), jnp.int32),
            pltpu.SemaphoreType.DMA,
            pltpu.SemaphoreType.DMA,
        ],
    )
    sums, aux = sc_call(xs, lab)

    out = pl.pallas_call(
        _tc_epilogue,
        out_shape=jax.ShapeDtypeStruct((1, 1), jnp.float32),
    )(sums, aux)
    return out[0, 0]
